# baseline (speedup 1.0000x reference)
import functools

import jax
import jax.numpy as jnp
from jax import lax
from jax.experimental import pallas as pl
from jax.experimental.pallas import tpu as pltpu
from jax.experimental.pallas import tpu_sc as plsc

F32 = jnp.float32
I32 = jnp.int32


def _a_body(x_ref, wv1_ref, wa_ref, wef_ref, xw_ref, xa_ref, y_ref):
    xb = x_ref[...]
    xw_ref[...] = jnp.dot(xb, wv1_ref[...], preferred_element_type=F32)
    xa_ref[...] = jnp.dot(xb, wa_ref[...], preferred_element_type=F32)
    y_ref[...] = jnp.dot(xb, wef_ref[...], preferred_element_type=F32)


def _precompute(x, wv1, wa, wef):
    n, d = x.shape
    nb = 1000
    grid = n // nb
    return pl.pallas_call(
        _a_body,
        grid=(grid,),
        in_specs=[
            pl.BlockSpec((nb, d), lambda i: (i, 0)),
            pl.BlockSpec(wv1.shape, lambda i: (0, 0)),
            pl.BlockSpec(wa.shape, lambda i: (0, 0)),
            pl.BlockSpec(wef.shape, lambda i: (0, 0)),
        ],
        out_specs=[
            pl.BlockSpec((nb, wv1.shape[1]), lambda i: (i, 0)),
            pl.BlockSpec((nb, wa.shape[1]), lambda i: (i, 0)),
            pl.BlockSpec((nb, wef.shape[1]), lambda i: (i, 0)),
        ],
        out_shape=[
            jax.ShapeDtypeStruct((n, wv1.shape[1]), F32),
            jax.ShapeDtypeStruct((n, wa.shape[1]), F32),
            jax.ShapeDtypeStruct((n, wef.shape[1]), F32),
        ],
    )(x, wv1, wa, wef)


def _b_body(adj_ref, xw_ref, wvs_ref, sp_ref, sf_ref, cnt_ref):
    i = pl.program_id(0)
    ab = adj_ref[...]
    deg = jnp.sum(ab, axis=1, keepdims=True)
    an = ab / (deg + 1e-8)
    h1 = jnp.maximum(jnp.dot(an, xw_ref[...], preferred_element_type=F32), 0.0)
    sl = jnp.dot(h1, wvs_ref[...], preferred_element_type=F32)
    pred = sl[:, 1:2] > sl[:, 0:1]
    sp_ref[...] = pred.astype(I32)
    predf = pred.astype(F32)
    sf_ref[...] = predf

    @pl.when(i == 0)
    def _():
        cnt_ref[...] = jnp.zeros((1, 1), F32)

    cnt_ref[...] += jnp.sum(predf, keepdims=True)


def _vgae(adj, xw, wvs):
    n = adj.shape[0]
    nb = 200
    grid = n // nb
    return pl.pallas_call(
        _b_body,
        grid=(grid,),
        in_specs=[
            pl.BlockSpec((nb, n), lambda i: (i, 0)),
            pl.BlockSpec(xw.shape, lambda i: (0, 0)),
            pl.BlockSpec(wvs.shape, lambda i: (0, 0)),
        ],
        out_specs=[
            pl.BlockSpec((nb, 1), lambda i: (i, 0)),
            pl.BlockSpec((nb, 1), lambda i: (i, 0)),
            pl.BlockSpec((1, 1), lambda i: (0, 0)),
        ],
        out_shape=[
            jax.ShapeDtypeStruct((n, 1), I32),
            jax.ShapeDtypeStruct((n, 1), F32),
            jax.ShapeDtypeStruct((1, 1), F32),
        ],
    )(adj, xw, wvs)


def _omega_kernel(n_nodes, ep):
    ew = ep // 32
    mesh = plsc.VectorSubcoreMesh(core_axis_name="c", subcore_axis_name="s")

    @functools.partial(
        pl.kernel,
        out_type=jax.ShapeDtypeStruct((ep * 4,), F32),
        mesh=mesh,
        compiler_params=pltpu.CompilerParams(needs_layout_passes=False),
        scratch_types=[
            pltpu.VMEM((n_nodes * 4,), F32),
            pltpu.VMEM((ew,), I32),
            pltpu.VMEM((ew,), I32),
            pltpu.VMEM((ew * 4,), F32),
        ],
    )
    def k(xa_hbm, src_hbm, dst_hbm, om_hbm, xav, srcv, dstv, omv):
        wid = lax.axis_index("s") * 2 + lax.axis_index("c")
        base = wid * ew
        pltpu.sync_copy(xa_hbm, xav)
        pltpu.sync_copy(src_hbm.at[pl.ds(base, ew)], srcv)
        pltpu.sync_copy(dst_hbm.at[pl.ds(base, ew)], dstv)
        lane = lax.iota(I32, 16)
        nmax = jnp.full((16,), n_nodes - 1, I32)

        def chunk(i, carry):
            sv = jnp.minimum(srcv[pl.ds(i * 16, 16)], nmax) * 4
            dv = jnp.minimum(dstv[pl.ds(i * 16, 16)], nmax) * 4
            z = []
            for c in range(4):
                colc = jnp.full((16,), c, I32)
                za = plsc.load_gather(xav, [sv + colc])
                zb = plsc.load_gather(xav, [dv + colc])
                z.append(za + zb)
            m = jnp.maximum(jnp.maximum(z[0], z[1]), jnp.maximum(z[2], z[3]))
            es = [jnp.exp(zc - m) for zc in z]
            tot = es[0] + es[1] + es[2] + es[3]
            inv = 1.0 / tot
            eidx = (jnp.full((16,), i * 16, I32) + lane) * 4
            for c in range(4):
                plsc.store_scatter(omv, [eidx + c], es[c] * inv)
            return carry

        lax.fori_loop(0, ew // 16, chunk, 0)
        pltpu.sync_copy(omv, om_hbm.at[pl.ds(base * 4, ew * 4)])

    return k


def _scatter_kernel(n_nodes, fw, ep):
    rpt = 320
    acc_rows = 328
    fb = 128
    cap = 256
    sck = 2048
    now = ep // sck
    mesh = plsc.VectorSubcoreMesh(core_axis_name="c", subcore_axis_name="s")

    @functools.partial(
        pl.kernel,
        out_type=jax.ShapeDtypeStruct((n_nodes * fw,), F32),
        mesh=mesh,
        compiler_params=pltpu.CompilerParams(needs_layout_passes=False),
        scratch_types=[
            pltpu.VMEM((acc_rows * fw,), F32),
            pltpu.VMEM((fb, fw), F32),
            pltpu.VMEM((sck,), I32),
            pltpu.VMEM((sck,), I32),
            pltpu.VMEM((cap,), I32),
            pltpu.VMEM((cap,), I32),
            pltpu.VMEM((cap,), I32),
            pltpu.VMEM((4, fb), I32),
            pltpu.VMEM((4 * fb,), F32),
            pltpu.SemaphoreType.DMA,
            pltpu.SemaphoreType.DMA,
        ],
    )
    def k(y_hbm, src_hbm, dst_hbm, om_hbm, out_hbm, acc, gbuf, srcw, dstw,
          csrc, clv, cpe, omidx, obuf, sem, sem2):
        w = lax.axis_index("s") * 2 + lax.axis_index("c")
        base_row = w * rpt
        lane = lax.iota(I32, 16)
        zero16 = jnp.zeros((16,), F32)
        trash16 = jnp.full((16,), rpt, I32) + (lane & 3)

        def zrow(i, carry):
            base = i * fw
            for v in range(fw // 16):
                acc[pl.ds(base + v * 16, 16)] = zero16
            return carry

        lax.fori_loop(0, acc_rows, zrow, 0)

        def flush():
            def omi(kk, cy):
                pe4 = cpe[pl.ds(kk * 16, 16)] * 4
                for ch in range(4):
                    omidx[ch, pl.ds(kk * 16, 16)] = pe4 + ch
                return cy

            lax.fori_loop(0, fb // 16, omi, 0)
            cpy = pltpu.async_copy(y_hbm.at[csrc.at[pl.ds(0, fb)]], gbuf,
                                   sem)
            cps = [pltpu.async_copy(om_hbm.at[omidx.at[ch]],
                                    obuf.at[pl.ds(ch * fb, fb)], sem2)
                   for ch in range(4)]
            cpy.wait()
            for cp in cps:
                cp.wait()

            def fedge(e, cy):
                ev = jnp.full((16,), e, I32)
                rowv = plsc.load_gather(clv, [ev])
                rowbase = rowv * fw + lane
                for ch in range(4):
                    wv = plsc.load_gather(obuf, [ev + (ch * fb)])
                    for v in range(4):
                        off = ch * 64 + v * 16
                        val = gbuf[e, pl.ds(off, 16)] * wv
                        plsc.addupdate_scatter(acc, [rowbase + off], val)
                return cy

            lax.fori_loop(0, fb, fedge, 0)

        def window(o, ptr):
            eb = o * sck
            pltpu.sync_copy(src_hbm.at[pl.ds(eb, sck)], srcw)
            pltpu.sync_copy(dst_hbm.at[pl.ds(eb, sck)], dstw)

            def block(b, ptr_in):
                datas = []
                cnts = []
                for q in range(4):
                    j_base = b * 64 + q * 16
                    s16 = srcw[pl.ds(j_base, 16)]
                    d16 = dstw[pl.ds(j_base, 16)]
                    lv = d16 - base_row
                    mask = (lv >= 0) & (lv < rpt)
                    pe = jnp.full((16,), eb, I32) + (j_base + lane)
                    datas.append((s16, lv, pe, mask))
                    cnts.append(plsc.all_reduce_population_count(mask)[0])
                p = ptr_in
                for q in range(4):
                    s16, lv, pe, mask = datas[q]
                    plsc.store_compressed(csrc.at[pl.ds(p, 16)], s16,
                                          mask=mask)
                    plsc.store_compressed(clv.at[pl.ds(p, 16)], lv,
                                          mask=mask)
                    plsc.store_compressed(cpe.at[pl.ds(p, 16)], pe,
                                          mask=mask)
                    p = p + cnts[q]

                @pl.when(p >= fb)
                def _():
                    flush()
                    for ref in (csrc, clv, cpe):
                        for q2 in range(4):
                            ref[pl.ds(q2 * 16, 16)] = \
                                ref[pl.ds(fb + q2 * 16, 16)]

                return lax.select(p >= fb, p - fb, p)

            return lax.fori_loop(0, sck // 64, block, ptr)

        ptr = lax.fori_loop(0, now, window, 0)

        for kk in range(fb // 16):
            csrc[pl.ds(ptr + kk * 16, 16)] = lane * 0
            clv[pl.ds(ptr + kk * 16, 16)] = trash16
            cpe[pl.ds(ptr + kk * 16, 16)] = lane * 0
        flush()

        @pl.when(w < 31)
        def _():
            pltpu.sync_copy(acc.at[pl.ds(0, rpt * fw)],
                            out_hbm.at[pl.ds(base_row * fw, rpt * fw)])

        @pl.when(w == 31)
        def _():
            last = (n_nodes - 31 * rpt) * fw
            pltpu.sync_copy(acc.at[pl.ds(0, last)],
                            out_hbm.at[pl.ds(base_row * fw, last)])

    return k


def _e_body(hp_ref, sf_ref, wd_ref, bd_ref, h_ref, clog_ref, lsum_ref):
    i = pl.program_id(0)
    hb = jnp.maximum(hp_ref[...], 0.0)
    h_ref[...] = hb
    wd = wd_ref[...]
    bd = bd_ref[...]
    s = sf_ref[...]
    parts = []
    for c in range(4):
        hc = hb[:, c * 64:(c + 1) * 64]
        lg = jnp.dot(hc, wd, preferred_element_type=F32) + bd
        clog_ref[c, :, :] = lg
        l0 = lg[:, 0:1]
        l1 = lg[:, 1:2]
        m = jnp.maximum(l0, l1)
        lse = m + jnp.log(jnp.exp(l0 - m) + jnp.exp(l1 - m))
        picked = jnp.where(s > 0.5, l1, l0)
        parts.append(jnp.sum(lse - picked, axis=0, keepdims=True))
    part = jnp.concatenate(parts, axis=1)

    @pl.when(i == 0)
    def _():
        lsum_ref[...] = jnp.zeros((1, 4), F32)

    lsum_ref[...] += part


def _heads(h_pre, s_f32, wd, bd):
    n, fw = h_pre.shape
    nb = 200
    grid = n // nb
    return pl.pallas_call(
        _e_body,
        grid=(grid,),
        in_specs=[
            pl.BlockSpec((nb, fw), lambda i: (i, 0)),
            pl.BlockSpec((nb, 1), lambda i: (i, 0)),
            pl.BlockSpec(wd.shape, lambda i: (0, 0)),
            pl.BlockSpec((1, 2), lambda i: (0, 0)),
        ],
        out_specs=[
            pl.BlockSpec((nb, fw), lambda i: (i, 0)),
            pl.BlockSpec((4, nb, 2), lambda i: (0, i, 0)),
            pl.BlockSpec((1, 4), lambda i: (0, 0)),
        ],
        out_shape=[
            jax.ShapeDtypeStruct((n, fw), F32),
            jax.ShapeDtypeStruct((4, n, 2), F32),
            jax.ShapeDtypeStruct((1, 4), F32),
        ],
    )(h_pre, s_f32, wd, bd)


def _f_body(h_ref, lsum_ref, cnt_ref, mp_ref, wc1_ref, bc1_ref, wc2_ref,
            bc2_ref, y_ref):
    n_total = 10000.0
    cl = lsum_ref[...] / n_total
    cw = cnt_ref[...]
    p1 = cw / n_total
    p0 = 1.0 - p1
    hs = -(p0 * jnp.log(p0 + 1e-12) + p1 * jnp.log(p1 + 1e-12))
    cs = jnp.clip(1.0 - cl / (hs + 1e-8), 0.0, 1.0)
    gate = jax.nn.sigmoid(mp_ref[...])
    hb = h_ref[...]
    acc = None
    for c in range(4):
        factor = 1.0 - cs[0, c] * gate[c:c + 1, :]
        hf = hb[:, c * 64:(c + 1) * 64] * factor
        t = jnp.dot(hf, wc1_ref[c], preferred_element_type=F32)
        acc = t if acc is None else acc + t
    t1 = jnp.maximum(acc + bc1_ref[...], 0.0)
    y_ref[...] = jnp.dot(t1, wc2_ref[...], preferred_element_type=F32) \
        + bc2_ref[...]


def _classifier(h, lsum, cnt, mp, wc1, bc1, wc2, bc2):
    n, fw = h.shape
    nb = 200
    grid = n // nb
    wc1r = wc1.reshape(4, 64, 64)
    return pl.pallas_call(
        _f_body,
        grid=(grid,),
        in_specs=[
            pl.BlockSpec((nb, fw), lambda i: (i, 0)),
            pl.BlockSpec((1, 4), lambda i: (0, 0)),
            pl.BlockSpec((1, 1), lambda i: (0, 0)),
            pl.BlockSpec(mp.shape, lambda i: (0, 0)),
            pl.BlockSpec(wc1r.shape, lambda i: (0, 0, 0)),
            pl.BlockSpec((1, 64), lambda i: (0, 0)),
            pl.BlockSpec(wc2.shape, lambda i: (0, 0)),
            pl.BlockSpec((1, 2), lambda i: (0, 0)),
        ],
        out_specs=pl.BlockSpec((nb, 2), lambda i: (i, 0)),
        out_shape=jax.ShapeDtypeStruct((n, 2), F32),
    )(h, lsum, cnt, mp, wc1r, bc1.reshape(1, 64), wc2, bc2.reshape(1, 2))


def kernel(adj, x, edge_index, W_vgae1, W_vgae_s, W_assign, W_enc, W_demo,
           b_demo, mask_param, W_cls1, b_cls1, W_cls2, b_cls2):
    n, d = x.shape
    c_ch, _, hdim = W_enc.shape
    fw = c_ch * hdim
    e = edge_index.shape[1]
    ep = ((e + 4095) // 4096) * 4096

    wef = jnp.transpose(W_enc, (1, 0, 2)).reshape(d, fw)
    xw, xa, y = _precompute(x, W_vgae1, W_assign, wef)
    spred2, s_f32, cnt = _vgae(adj, xw, W_vgae_s)

    src = jnp.concatenate([edge_index[0], jnp.zeros((ep - e,), I32)])
    dst = jnp.concatenate([edge_index[1], jnp.full((ep - e,), n, I32)])

    om = _omega_kernel(n, ep)(xa.reshape(-1), src, dst)
    h_pre = _scatter_kernel(n, fw, ep)(y, src, dst, om).reshape(n, fw)

    h_out, clog, lsum = _heads(h_pre, s_f32, W_demo, b_demo.reshape(1, 2))
    y_logits = _classifier(h_out, lsum, cnt, mask_param, W_cls1, b_cls1,
                           W_cls2, b_cls2)

    h = h_out.reshape(n, c_ch, hdim)
    s_pred = spred2[:, 0]
    return y_logits, h, clog, s_pred


# speedup vs baseline: 1.0836x; 1.0836x over previous
import functools

import jax
import jax.numpy as jnp
from jax import lax
from jax.experimental import pallas as pl
from jax.experimental.pallas import tpu as pltpu
from jax.experimental.pallas import tpu_sc as plsc

F32 = jnp.float32
I32 = jnp.int32


def _a_body(x_ref, wv1_ref, wa_ref, wef_ref, xw_ref, xa_ref, y_ref):
    xb = x_ref[...]
    xw_ref[...] = jnp.dot(xb, wv1_ref[...], preferred_element_type=F32)
    xa_ref[...] = jnp.dot(xb, wa_ref[...], preferred_element_type=F32)
    y_ref[...] = jnp.dot(xb, wef_ref[...], preferred_element_type=F32)


def _precompute(x, wv1, wa, wef):
    n, d = x.shape
    nb = 1000
    grid = n // nb
    return pl.pallas_call(
        _a_body,
        grid=(grid,),
        in_specs=[
            pl.BlockSpec((nb, d), lambda i: (i, 0)),
            pl.BlockSpec(wv1.shape, lambda i: (0, 0)),
            pl.BlockSpec(wa.shape, lambda i: (0, 0)),
            pl.BlockSpec(wef.shape, lambda i: (0, 0)),
        ],
        out_specs=[
            pl.BlockSpec((nb, wv1.shape[1]), lambda i: (i, 0)),
            pl.BlockSpec((nb, wa.shape[1]), lambda i: (i, 0)),
            pl.BlockSpec((nb, wef.shape[1]), lambda i: (i, 0)),
        ],
        out_shape=[
            jax.ShapeDtypeStruct((n, wv1.shape[1]), F32),
            jax.ShapeDtypeStruct((n, wa.shape[1]), F32),
            jax.ShapeDtypeStruct((n, wef.shape[1]), F32),
        ],
    )(x, wv1, wa, wef)


def _b_body(adj_ref, xw_ref, wvs_ref, sp_ref, sf_ref, cnt_ref):
    i = pl.program_id(0)
    ab = adj_ref[...]
    deg = jnp.sum(ab, axis=1, keepdims=True)
    an = ab / (deg + 1e-8)
    h1 = jnp.maximum(jnp.dot(an, xw_ref[...], preferred_element_type=F32), 0.0)
    sl = jnp.dot(h1, wvs_ref[...], preferred_element_type=F32)
    pred = sl[:, 1:2] > sl[:, 0:1]
    sp_ref[...] = pred.astype(I32)
    predf = pred.astype(F32)
    sf_ref[...] = predf

    @pl.when(i == 0)
    def _():
        cnt_ref[...] = jnp.zeros((1, 1), F32)

    cnt_ref[...] += jnp.sum(predf, keepdims=True)


def _vgae(adj, xw, wvs):
    n = adj.shape[0]
    nb = 200
    grid = n // nb
    return pl.pallas_call(
        _b_body,
        grid=(grid,),
        in_specs=[
            pl.BlockSpec((nb, n), lambda i: (i, 0)),
            pl.BlockSpec(xw.shape, lambda i: (0, 0)),
            pl.BlockSpec(wvs.shape, lambda i: (0, 0)),
        ],
        out_specs=[
            pl.BlockSpec((nb, 1), lambda i: (i, 0)),
            pl.BlockSpec((nb, 1), lambda i: (i, 0)),
            pl.BlockSpec((1, 1), lambda i: (0, 0)),
        ],
        out_shape=[
            jax.ShapeDtypeStruct((n, 1), I32),
            jax.ShapeDtypeStruct((n, 1), F32),
            jax.ShapeDtypeStruct((1, 1), F32),
        ],
    )(adj, xw, wvs)


def _omega_kernel(n_nodes, ep):
    ew = ep // 32
    mesh = plsc.VectorSubcoreMesh(core_axis_name="c", subcore_axis_name="s")

    @functools.partial(
        pl.kernel,
        out_type=jax.ShapeDtypeStruct((ep * 4,), F32),
        mesh=mesh,
        compiler_params=pltpu.CompilerParams(needs_layout_passes=False),
        scratch_types=[
            pltpu.VMEM((n_nodes * 4,), F32),
            pltpu.VMEM((ew,), I32),
            pltpu.VMEM((ew,), I32),
            pltpu.VMEM((ew * 4,), F32),
        ],
    )
    def k(xa_hbm, src_hbm, dst_hbm, om_hbm, xav, srcv, dstv, omv):
        wid = lax.axis_index("s") * 2 + lax.axis_index("c")
        base = wid * ew
        pltpu.sync_copy(xa_hbm, xav)
        pltpu.sync_copy(src_hbm.at[pl.ds(base, ew)], srcv)
        pltpu.sync_copy(dst_hbm.at[pl.ds(base, ew)], dstv)
        lane = lax.iota(I32, 16)
        nmax = jnp.full((16,), n_nodes - 1, I32)

        def chunk(i, carry):
            sv = jnp.minimum(srcv[pl.ds(i * 16, 16)], nmax) * 4
            dv = jnp.minimum(dstv[pl.ds(i * 16, 16)], nmax) * 4
            z = []
            for c in range(4):
                colc = jnp.full((16,), c, I32)
                za = plsc.load_gather(xav, [sv + colc])
                zb = plsc.load_gather(xav, [dv + colc])
                z.append(za + zb)
            m = jnp.maximum(jnp.maximum(z[0], z[1]), jnp.maximum(z[2], z[3]))
            es = [jnp.exp(zc - m) for zc in z]
            tot = es[0] + es[1] + es[2] + es[3]
            inv = 1.0 / tot
            eidx = (jnp.full((16,), i * 16, I32) + lane) * 4
            for c in range(4):
                plsc.store_scatter(omv, [eidx + c], es[c] * inv)
            return carry

        lax.fori_loop(0, ew // 16, chunk, 0)
        pltpu.sync_copy(omv, om_hbm.at[pl.ds(base * 4, ew * 4)])

    return k


def _scatter_kernel(n_nodes, fw, ep):
    rpt = 320
    acc_rows = 328
    fb = 128
    cap = 256
    sck = 2048
    now = ep // sck
    mesh = plsc.VectorSubcoreMesh(core_axis_name="c", subcore_axis_name="s")

    @functools.partial(
        pl.kernel,
        out_type=jax.ShapeDtypeStruct((n_nodes * fw,), F32),
        mesh=mesh,
        compiler_params=pltpu.CompilerParams(needs_layout_passes=False),
        scratch_types=[
            pltpu.VMEM((acc_rows * fw,), F32),
            pltpu.VMEM((fb, fw), F32),
            pltpu.VMEM((2, sck), I32),
            pltpu.VMEM((2, sck), I32),
            pltpu.VMEM((cap,), I32),
            pltpu.VMEM((cap,), I32),
            pltpu.VMEM((cap,), I32),
            pltpu.VMEM((4, fb), I32),
            pltpu.VMEM((4 * fb,), F32),
            pltpu.SemaphoreType.DMA,
            pltpu.SemaphoreType.DMA,
            pltpu.SemaphoreType.DMA,
        ],
    )
    def k(y_hbm, src_hbm, dst_hbm, om_hbm, out_hbm, acc, gbuf, srcw, dstw,
          csrc, clv, cpe, omidx, obuf, sem, sem2, sems):
        w = lax.axis_index("s") * 2 + lax.axis_index("c")
        base_row = w * rpt
        lane = lax.iota(I32, 16)
        zero16 = jnp.zeros((16,), F32)
        trash16 = jnp.full((16,), rpt, I32) + (lane & 3)

        def zrow(i, carry):
            base = i * fw
            for v in range(fw // 16):
                acc[pl.ds(base + v * 16, 16)] = zero16
            return carry

        lax.fori_loop(0, acc_rows, zrow, 0)

        def flush():
            def omi(kk, cy):
                pe4 = cpe[pl.ds(kk * 16, 16)] * 4
                for ch in range(4):
                    omidx[ch, pl.ds(kk * 16, 16)] = pe4 + ch
                return cy

            lax.fori_loop(0, fb // 16, omi, 0)
            qs = fb // 4
            cpys = [pltpu.async_copy(
                y_hbm.at[csrc.at[pl.ds(q * qs, qs)]],
                gbuf.at[pl.ds(q * qs, qs)], sem) for q in range(4)]
            cps = [pltpu.async_copy(om_hbm.at[omidx.at[ch]],
                                    obuf.at[pl.ds(ch * fb, fb)], sem2)
                   for ch in range(4)]
            for cp in cpys:
                cp.wait()
            for cp in cps:
                cp.wait()

            def fedge(e, cy):
                ev = jnp.full((16,), e, I32)
                rowv = plsc.load_gather(clv, [ev])
                rowbase = rowv * fw + lane
                for ch in range(4):
                    wv = plsc.load_gather(obuf, [ev + (ch * fb)])
                    for v in range(4):
                        off = ch * 64 + v * 16
                        val = gbuf[e, pl.ds(off, 16)] * wv
                        plsc.addupdate_scatter(acc, [rowbase + off], val)
                return cy

            lax.fori_loop(0, fb, fedge, 0, unroll=2)

        pltpu.async_copy(src_hbm.at[pl.ds(0, sck)], srcw.at[0], sems)
        pltpu.async_copy(dst_hbm.at[pl.ds(0, sck)], dstw.at[0], sems)

        def window(o, ptr):
            eb = o * sck
            par = o % 2
            pltpu.make_async_copy(src_hbm.at[pl.ds(eb, sck)],
                                  srcw.at[par], sems).wait()
            pltpu.make_async_copy(dst_hbm.at[pl.ds(eb, sck)],
                                  dstw.at[par], sems).wait()

            @pl.when(o + 1 < now)
            def _():
                nxt = (o + 1) % 2
                eb2 = (o + 1) * sck
                pltpu.async_copy(src_hbm.at[pl.ds(eb2, sck)],
                                 srcw.at[nxt], sems)
                pltpu.async_copy(dst_hbm.at[pl.ds(eb2, sck)],
                                 dstw.at[nxt], sems)

            def block(b, ptr_in):
                datas = []
                cnts = []
                for q in range(4):
                    j_base = b * 64 + q * 16
                    s16 = srcw[par, pl.ds(j_base, 16)]
                    d16 = dstw[par, pl.ds(j_base, 16)]
                    lv = d16 - base_row
                    mask = (lv >= 0) & (lv < rpt)
                    pe = jnp.full((16,), eb, I32) + (j_base + lane)
                    datas.append((s16, lv, pe, mask))
                    cnts.append(plsc.all_reduce_population_count(mask)[0])
                p = ptr_in
                for q in range(4):
                    s16, lv, pe, mask = datas[q]
                    plsc.store_compressed(csrc.at[pl.ds(p, 16)], s16,
                                          mask=mask)
                    plsc.store_compressed(clv.at[pl.ds(p, 16)], lv,
                                          mask=mask)
                    plsc.store_compressed(cpe.at[pl.ds(p, 16)], pe,
                                          mask=mask)
                    p = p + cnts[q]

                @pl.when(p >= fb)
                def _():
                    flush()
                    for ref in (csrc, clv, cpe):
                        for q2 in range(4):
                            ref[pl.ds(q2 * 16, 16)] = \
                                ref[pl.ds(fb + q2 * 16, 16)]

                return lax.select(p >= fb, p - fb, p)

            return lax.fori_loop(0, sck // 64, block, ptr)

        ptr = lax.fori_loop(0, now, window, 0)

        for kk in range(fb // 16):
            csrc[pl.ds(ptr + kk * 16, 16)] = lane * 0
            clv[pl.ds(ptr + kk * 16, 16)] = trash16
            cpe[pl.ds(ptr + kk * 16, 16)] = lane * 0
        flush()

        @pl.when(w < 31)
        def _():
            pltpu.sync_copy(acc.at[pl.ds(0, rpt * fw)],
                            out_hbm.at[pl.ds(base_row * fw, rpt * fw)])

        @pl.when(w == 31)
        def _():
            last = (n_nodes - 31 * rpt) * fw
            pltpu.sync_copy(acc.at[pl.ds(0, last)],
                            out_hbm.at[pl.ds(base_row * fw, last)])

    return k


def _e_body(hp_ref, sf_ref, wd_ref, bd_ref, h_ref, clog_ref, lsum_ref):
    i = pl.program_id(0)
    hb = jnp.maximum(hp_ref[...], 0.0)
    h_ref[...] = hb
    wd = wd_ref[...]
    bd = bd_ref[...]
    s = sf_ref[...]
    parts = []
    for c in range(4):
        hc = hb[:, c * 64:(c + 1) * 64]
        lg = jnp.dot(hc, wd, preferred_element_type=F32) + bd
        clog_ref[c, :, :] = lg
        l0 = lg[:, 0:1]
        l1 = lg[:, 1:2]
        m = jnp.maximum(l0, l1)
        lse = m + jnp.log(jnp.exp(l0 - m) + jnp.exp(l1 - m))
        picked = jnp.where(s > 0.5, l1, l0)
        parts.append(jnp.sum(lse - picked, axis=0, keepdims=True))
    part = jnp.concatenate(parts, axis=1)

    @pl.when(i == 0)
    def _():
        lsum_ref[...] = jnp.zeros((1, 4), F32)

    lsum_ref[...] += part


def _heads(h_pre, s_f32, wd, bd):
    n, fw = h_pre.shape
    nb = 200
    grid = n // nb
    return pl.pallas_call(
        _e_body,
        grid=(grid,),
        in_specs=[
            pl.BlockSpec((nb, fw), lambda i: (i, 0)),
            pl.BlockSpec((nb, 1), lambda i: (i, 0)),
            pl.BlockSpec(wd.shape, lambda i: (0, 0)),
            pl.BlockSpec((1, 2), lambda i: (0, 0)),
        ],
        out_specs=[
            pl.BlockSpec((nb, fw), lambda i: (i, 0)),
            pl.BlockSpec((4, nb, 2), lambda i: (0, i, 0)),
            pl.BlockSpec((1, 4), lambda i: (0, 0)),
        ],
        out_shape=[
            jax.ShapeDtypeStruct((n, fw), F32),
            jax.ShapeDtypeStruct((4, n, 2), F32),
            jax.ShapeDtypeStruct((1, 4), F32),
        ],
    )(h_pre, s_f32, wd, bd)


def _f_body(h_ref, lsum_ref, cnt_ref, mp_ref, wc1_ref, bc1_ref, wc2_ref,
            bc2_ref, y_ref):
    n_total = 10000.0
    cl = lsum_ref[...] / n_total
    cw = cnt_ref[...]
    p1 = cw / n_total
    p0 = 1.0 - p1
    hs = -(p0 * jnp.log(p0 + 1e-12) + p1 * jnp.log(p1 + 1e-12))
    cs = jnp.clip(1.0 - cl / (hs + 1e-8), 0.0, 1.0)
    gate = jax.nn.sigmoid(mp_ref[...])
    hb = h_ref[...]
    acc = None
    for c in range(4):
        factor = 1.0 - cs[0, c] * gate[c:c + 1, :]
        hf = hb[:, c * 64:(c + 1) * 64] * factor
        t = jnp.dot(hf, wc1_ref[c], preferred_element_type=F32)
        acc = t if acc is None else acc + t
    t1 = jnp.maximum(acc + bc1_ref[...], 0.0)
    y_ref[...] = jnp.dot(t1, wc2_ref[...], preferred_element_type=F32) \
        + bc2_ref[...]


def _classifier(h, lsum, cnt, mp, wc1, bc1, wc2, bc2):
    n, fw = h.shape
    nb = 200
    grid = n // nb
    wc1r = wc1.reshape(4, 64, 64)
    return pl.pallas_call(
        _f_body,
        grid=(grid,),
        in_specs=[
            pl.BlockSpec((nb, fw), lambda i: (i, 0)),
            pl.BlockSpec((1, 4), lambda i: (0, 0)),
            pl.BlockSpec((1, 1), lambda i: (0, 0)),
            pl.BlockSpec(mp.shape, lambda i: (0, 0)),
            pl.BlockSpec(wc1r.shape, lambda i: (0, 0, 0)),
            pl.BlockSpec((1, 64), lambda i: (0, 0)),
            pl.BlockSpec(wc2.shape, lambda i: (0, 0)),
            pl.BlockSpec((1, 2), lambda i: (0, 0)),
        ],
        out_specs=pl.BlockSpec((nb, 2), lambda i: (i, 0)),
        out_shape=jax.ShapeDtypeStruct((n, 2), F32),
    )(h, lsum, cnt, mp, wc1r, bc1.reshape(1, 64), wc2, bc2.reshape(1, 2))


def kernel(adj, x, edge_index, W_vgae1, W_vgae_s, W_assign, W_enc, W_demo,
           b_demo, mask_param, W_cls1, b_cls1, W_cls2, b_cls2):
    n, d = x.shape
    c_ch, _, hdim = W_enc.shape
    fw = c_ch * hdim
    e = edge_index.shape[1]
    ep = ((e + 4095) // 4096) * 4096

    wef = jnp.transpose(W_enc, (1, 0, 2)).reshape(d, fw)
    xw, xa, y = _precompute(x, W_vgae1, W_assign, wef)
    spred2, s_f32, cnt = _vgae(adj, xw, W_vgae_s)

    src = jnp.concatenate([edge_index[0], jnp.zeros((ep - e,), I32)])
    dst = jnp.concatenate([edge_index[1], jnp.full((ep - e,), n, I32)])

    om = _omega_kernel(n, ep)(xa.reshape(-1), src, dst)
    h_pre = _scatter_kernel(n, fw, ep)(y, src, dst, om).reshape(n, fw)

    h_out, clog, lsum = _heads(h_pre, s_f32, W_demo, b_demo.reshape(1, 2))
    y_logits = _classifier(h_out, lsum, cnt, mask_param, W_cls1, b_cls1,
                           W_cls2, b_cls2)

    h = h_out.reshape(n, c_ch, hdim)
    s_pred = spred2[:, 0]
    return y_logits, h, clog, s_pred


# speedup vs baseline: 1.0869x; 1.0031x over previous
import functools

import jax
import jax.numpy as jnp
from jax import lax
from jax.experimental import pallas as pl
from jax.experimental.pallas import tpu as pltpu
from jax.experimental.pallas import tpu_sc as plsc

F32 = jnp.float32
I32 = jnp.int32


def _a_body(x_ref, wv1_ref, wa_ref, wef_ref, xw_ref, xa_ref, y_ref):
    xb = x_ref[...]
    xw_ref[...] = jnp.dot(xb, wv1_ref[...], preferred_element_type=F32)
    xa_ref[...] = jnp.dot(xb, wa_ref[...], preferred_element_type=F32)
    y_ref[...] = jnp.dot(xb, wef_ref[...], preferred_element_type=F32)


def _precompute(x, wv1, wa, wef):
    n, d = x.shape
    nb = 1000
    grid = n // nb
    return pl.pallas_call(
        _a_body,
        grid=(grid,),
        in_specs=[
            pl.BlockSpec((nb, d), lambda i: (i, 0)),
            pl.BlockSpec(wv1.shape, lambda i: (0, 0)),
            pl.BlockSpec(wa.shape, lambda i: (0, 0)),
            pl.BlockSpec(wef.shape, lambda i: (0, 0)),
        ],
        out_specs=[
            pl.BlockSpec((nb, wv1.shape[1]), lambda i: (i, 0)),
            pl.BlockSpec((nb, wa.shape[1]), lambda i: (i, 0)),
            pl.BlockSpec((nb, wef.shape[1]), lambda i: (i, 0)),
        ],
        out_shape=[
            jax.ShapeDtypeStruct((n, wv1.shape[1]), F32),
            jax.ShapeDtypeStruct((n, wa.shape[1]), F32),
            jax.ShapeDtypeStruct((n, wef.shape[1]), F32),
        ],
    )(x, wv1, wa, wef)


def _b_body(adj_ref, xw_ref, wvs_ref, sp_ref, sf_ref, cnt_ref):
    i = pl.program_id(0)
    ab = adj_ref[...]
    deg = jnp.sum(ab, axis=1, keepdims=True)
    an = ab / (deg + 1e-8)
    h1 = jnp.maximum(jnp.dot(an, xw_ref[...], preferred_element_type=F32), 0.0)
    sl = jnp.dot(h1, wvs_ref[...], preferred_element_type=F32)
    pred = sl[:, 1:2] > sl[:, 0:1]
    sp_ref[...] = pred.astype(I32)
    predf = pred.astype(F32)
    sf_ref[...] = predf

    @pl.when(i == 0)
    def _():
        cnt_ref[...] = jnp.zeros((1, 1), F32)

    cnt_ref[...] += jnp.sum(predf, keepdims=True)


def _vgae(adj, xw, wvs):
    n = adj.shape[0]
    nb = 200
    grid = n // nb
    return pl.pallas_call(
        _b_body,
        grid=(grid,),
        in_specs=[
            pl.BlockSpec((nb, n), lambda i: (i, 0)),
            pl.BlockSpec(xw.shape, lambda i: (0, 0)),
            pl.BlockSpec(wvs.shape, lambda i: (0, 0)),
        ],
        out_specs=[
            pl.BlockSpec((nb, 1), lambda i: (i, 0)),
            pl.BlockSpec((nb, 1), lambda i: (i, 0)),
            pl.BlockSpec((1, 1), lambda i: (0, 0)),
        ],
        out_shape=[
            jax.ShapeDtypeStruct((n, 1), I32),
            jax.ShapeDtypeStruct((n, 1), F32),
            jax.ShapeDtypeStruct((1, 1), F32),
        ],
    )(adj, xw, wvs)


def _omega_kernel(n_nodes, ep):
    ew = ep // 32
    mesh = plsc.VectorSubcoreMesh(core_axis_name="c", subcore_axis_name="s")

    @functools.partial(
        pl.kernel,
        out_type=jax.ShapeDtypeStruct((ep * 4,), F32),
        mesh=mesh,
        compiler_params=pltpu.CompilerParams(needs_layout_passes=False),
        scratch_types=[
            pltpu.VMEM((n_nodes * 4,), F32),
            pltpu.VMEM((ew,), I32),
            pltpu.VMEM((ew,), I32),
            pltpu.VMEM((ew * 4,), F32),
        ],
    )
    def k(xa_hbm, src_hbm, dst_hbm, om_hbm, xav, srcv, dstv, omv):
        wid = lax.axis_index("s") * 2 + lax.axis_index("c")
        base = wid * ew
        pltpu.sync_copy(xa_hbm, xav)
        pltpu.sync_copy(src_hbm.at[pl.ds(base, ew)], srcv)
        pltpu.sync_copy(dst_hbm.at[pl.ds(base, ew)], dstv)
        lane = lax.iota(I32, 16)
        nmax = jnp.full((16,), n_nodes - 1, I32)

        def chunk(i, carry):
            sv = jnp.minimum(srcv[pl.ds(i * 16, 16)], nmax) * 4
            dv = jnp.minimum(dstv[pl.ds(i * 16, 16)], nmax) * 4
            z = []
            for c in range(4):
                colc = jnp.full((16,), c, I32)
                za = plsc.load_gather(xav, [sv + colc])
                zb = plsc.load_gather(xav, [dv + colc])
                z.append(za + zb)
            m = jnp.maximum(jnp.maximum(z[0], z[1]), jnp.maximum(z[2], z[3]))
            es = [jnp.exp(zc - m) for zc in z]
            tot = es[0] + es[1] + es[2] + es[3]
            inv = 1.0 / tot
            eidx = (jnp.full((16,), i * 16, I32) + lane) * 4
            for c in range(4):
                plsc.store_scatter(omv, [eidx + c], es[c] * inv)
            return carry

        lax.fori_loop(0, ew // 16, chunk, 0)
        pltpu.sync_copy(omv, om_hbm.at[pl.ds(base * 4, ew * 4)])

    return k


def _scatter_kernel(n_nodes, fw, ep):
    rpt = 320
    acc_rows = 328
    fb = 128
    cap = 256
    sck = 2048
    now = ep // sck
    mesh = plsc.VectorSubcoreMesh(core_axis_name="c", subcore_axis_name="s")

    @functools.partial(
        pl.kernel,
        out_type=jax.ShapeDtypeStruct((n_nodes * fw,), F32),
        mesh=mesh,
        compiler_params=pltpu.CompilerParams(needs_layout_passes=False),
        scratch_types=[
            pltpu.VMEM((acc_rows * fw,), F32),
            pltpu.VMEM((fb, fw), F32),
            pltpu.VMEM((2, sck), I32),
            pltpu.VMEM((2, sck), I32),
            pltpu.VMEM((cap,), I32),
            pltpu.VMEM((cap,), I32),
            pltpu.VMEM((cap,), I32),
            pltpu.VMEM((4, fb), I32),
            pltpu.VMEM((4 * fb,), F32),
            pltpu.SemaphoreType.DMA,
            pltpu.SemaphoreType.DMA,
            pltpu.SemaphoreType.DMA,
        ],
    )
    def k(y_hbm, src_hbm, dst_hbm, om_hbm, out_hbm, acc, gbuf, srcw, dstw,
          csrc, clv, cpe, omidx, obuf, sem, sem2, sems):
        w = lax.axis_index("s") * 2 + lax.axis_index("c")
        base_row = w * rpt
        lane = lax.iota(I32, 16)
        zero16 = jnp.zeros((16,), F32)
        trash16 = jnp.full((16,), rpt, I32) + (lane & 3)

        def zrow(i, carry):
            base = i * fw
            for v in range(fw // 16):
                acc[pl.ds(base + v * 16, 16)] = zero16
            return carry

        lax.fori_loop(0, acc_rows, zrow, 0)

        def flush():
            def omi(kk, cy):
                pe4 = cpe[pl.ds(kk * 16, 16)] * 4
                for ch in range(4):
                    omidx[ch, pl.ds(kk * 16, 16)] = pe4 + ch
                return cy

            lax.fori_loop(0, fb // 16, omi, 0)
            qs = fb // 8
            cpys = [pltpu.async_copy(
                y_hbm.at[csrc.at[pl.ds(q * qs, qs)]],
                gbuf.at[pl.ds(q * qs, qs)], sem) for q in range(8)]
            cps = [pltpu.async_copy(om_hbm.at[omidx.at[ch]],
                                    obuf.at[pl.ds(ch * fb, fb)], sem2)
                   for ch in range(4)]
            for cp in cpys:
                cp.wait()
            for cp in cps:
                cp.wait()

            def fedge(e, cy):
                ev = jnp.full((16,), e, I32)
                rowv = plsc.load_gather(clv, [ev])
                rowbase = rowv * fw + lane
                for ch in range(4):
                    wv = plsc.load_gather(obuf, [ev + (ch * fb)])
                    for v in range(4):
                        off = ch * 64 + v * 16
                        val = gbuf[e, pl.ds(off, 16)] * wv
                        plsc.addupdate_scatter(acc, [rowbase + off], val)
                return cy

            lax.fori_loop(0, fb, fedge, 0, unroll=4)

        pltpu.async_copy(src_hbm.at[pl.ds(0, sck)], srcw.at[0], sems)
        pltpu.async_copy(dst_hbm.at[pl.ds(0, sck)], dstw.at[0], sems)

        def window(o, ptr):
            eb = o * sck
            par = o % 2
            pltpu.make_async_copy(src_hbm.at[pl.ds(eb, sck)],
                                  srcw.at[par], sems).wait()
            pltpu.make_async_copy(dst_hbm.at[pl.ds(eb, sck)],
                                  dstw.at[par], sems).wait()

            @pl.when(o + 1 < now)
            def _():
                nxt = (o + 1) % 2
                eb2 = (o + 1) * sck
                pltpu.async_copy(src_hbm.at[pl.ds(eb2, sck)],
                                 srcw.at[nxt], sems)
                pltpu.async_copy(dst_hbm.at[pl.ds(eb2, sck)],
                                 dstw.at[nxt], sems)

            def block(b, ptr_in):
                datas = []
                cnts = []
                for q in range(4):
                    j_base = b * 64 + q * 16
                    s16 = srcw[par, pl.ds(j_base, 16)]
                    d16 = dstw[par, pl.ds(j_base, 16)]
                    lv = d16 - base_row
                    mask = (lv >= 0) & (lv < rpt)
                    pe = jnp.full((16,), eb, I32) + (j_base + lane)
                    datas.append((s16, lv, pe, mask))
                    cnts.append(plsc.all_reduce_population_count(mask)[0])
                p = ptr_in
                for q in range(4):
                    s16, lv, pe, mask = datas[q]
                    plsc.store_compressed(csrc.at[pl.ds(p, 16)], s16,
                                          mask=mask)
                    plsc.store_compressed(clv.at[pl.ds(p, 16)], lv,
                                          mask=mask)
                    plsc.store_compressed(cpe.at[pl.ds(p, 16)], pe,
                                          mask=mask)
                    p = p + cnts[q]

                @pl.when(p >= fb)
                def _():
                    flush()
                    for ref in (csrc, clv, cpe):
                        for q2 in range(4):
                            ref[pl.ds(q2 * 16, 16)] = \
                                ref[pl.ds(fb + q2 * 16, 16)]

                return lax.select(p >= fb, p - fb, p)

            return lax.fori_loop(0, sck // 64, block, ptr)

        ptr = lax.fori_loop(0, now, window, 0)

        for kk in range(fb // 16):
            csrc[pl.ds(ptr + kk * 16, 16)] = lane * 0
            clv[pl.ds(ptr + kk * 16, 16)] = trash16
            cpe[pl.ds(ptr + kk * 16, 16)] = lane * 0
        flush()

        @pl.when(w < 31)
        def _():
            pltpu.sync_copy(acc.at[pl.ds(0, rpt * fw)],
                            out_hbm.at[pl.ds(base_row * fw, rpt * fw)])

        @pl.when(w == 31)
        def _():
            last = (n_nodes - 31 * rpt) * fw
            pltpu.sync_copy(acc.at[pl.ds(0, last)],
                            out_hbm.at[pl.ds(base_row * fw, last)])

    return k


def _e_body(hp_ref, sf_ref, wd_ref, bd_ref, h_ref, clog_ref, lsum_ref):
    i = pl.program_id(0)
    hb = jnp.maximum(hp_ref[...], 0.0)
    h_ref[...] = hb
    wd = wd_ref[...]
    bd = bd_ref[...]
    s = sf_ref[...]
    parts = []
    for c in range(4):
        hc = hb[:, c * 64:(c + 1) * 64]
        lg = jnp.dot(hc, wd, preferred_element_type=F32) + bd
        clog_ref[c, :, :] = lg
        l0 = lg[:, 0:1]
        l1 = lg[:, 1:2]
        m = jnp.maximum(l0, l1)
        lse = m + jnp.log(jnp.exp(l0 - m) + jnp.exp(l1 - m))
        picked = jnp.where(s > 0.5, l1, l0)
        parts.append(jnp.sum(lse - picked, axis=0, keepdims=True))
    part = jnp.concatenate(parts, axis=1)

    @pl.when(i == 0)
    def _():
        lsum_ref[...] = jnp.zeros((1, 4), F32)

    lsum_ref[...] += part


def _heads(h_pre, s_f32, wd, bd):
    n, fw = h_pre.shape
    nb = 200
    grid = n // nb
    return pl.pallas_call(
        _e_body,
        grid=(grid,),
        in_specs=[
            pl.BlockSpec((nb, fw), lambda i: (i, 0)),
            pl.BlockSpec((nb, 1), lambda i: (i, 0)),
            pl.BlockSpec(wd.shape, lambda i: (0, 0)),
            pl.BlockSpec((1, 2), lambda i: (0, 0)),
        ],
        out_specs=[
            pl.BlockSpec((nb, fw), lambda i: (i, 0)),
            pl.BlockSpec((4, nb, 2), lambda i: (0, i, 0)),
            pl.BlockSpec((1, 4), lambda i: (0, 0)),
        ],
        out_shape=[
            jax.ShapeDtypeStruct((n, fw), F32),
            jax.ShapeDtypeStruct((4, n, 2), F32),
            jax.ShapeDtypeStruct((1, 4), F32),
        ],
    )(h_pre, s_f32, wd, bd)


def _f_body(h_ref, lsum_ref, cnt_ref, mp_ref, wc1_ref, bc1_ref, wc2_ref,
            bc2_ref, y_ref):
    n_total = 10000.0
    cl = lsum_ref[...] / n_total
    cw = cnt_ref[...]
    p1 = cw / n_total
    p0 = 1.0 - p1
    hs = -(p0 * jnp.log(p0 + 1e-12) + p1 * jnp.log(p1 + 1e-12))
    cs = jnp.clip(1.0 - cl / (hs + 1e-8), 0.0, 1.0)
    gate = jax.nn.sigmoid(mp_ref[...])
    hb = h_ref[...]
    acc = None
    for c in range(4):
        factor = 1.0 - cs[0, c] * gate[c:c + 1, :]
        hf = hb[:, c * 64:(c + 1) * 64] * factor
        t = jnp.dot(hf, wc1_ref[c], preferred_element_type=F32)
        acc = t if acc is None else acc + t
    t1 = jnp.maximum(acc + bc1_ref[...], 0.0)
    y_ref[...] = jnp.dot(t1, wc2_ref[...], preferred_element_type=F32) \
        + bc2_ref[...]


def _classifier(h, lsum, cnt, mp, wc1, bc1, wc2, bc2):
    n, fw = h.shape
    nb = 200
    grid = n // nb
    wc1r = wc1.reshape(4, 64, 64)
    return pl.pallas_call(
        _f_body,
        grid=(grid,),
        in_specs=[
            pl.BlockSpec((nb, fw), lambda i: (i, 0)),
            pl.BlockSpec((1, 4), lambda i: (0, 0)),
            pl.BlockSpec((1, 1), lambda i: (0, 0)),
            pl.BlockSpec(mp.shape, lambda i: (0, 0)),
            pl.BlockSpec(wc1r.shape, lambda i: (0, 0, 0)),
            pl.BlockSpec((1, 64), lambda i: (0, 0)),
            pl.BlockSpec(wc2.shape, lambda i: (0, 0)),
            pl.BlockSpec((1, 2), lambda i: (0, 0)),
        ],
        out_specs=pl.BlockSpec((nb, 2), lambda i: (i, 0)),
        out_shape=jax.ShapeDtypeStruct((n, 2), F32),
    )(h, lsum, cnt, mp, wc1r, bc1.reshape(1, 64), wc2, bc2.reshape(1, 2))


def kernel(adj, x, edge_index, W_vgae1, W_vgae_s, W_assign, W_enc, W_demo,
           b_demo, mask_param, W_cls1, b_cls1, W_cls2, b_cls2):
    n, d = x.shape
    c_ch, _, hdim = W_enc.shape
    fw = c_ch * hdim
    e = edge_index.shape[1]
    ep = ((e + 4095) // 4096) * 4096

    wef = jnp.transpose(W_enc, (1, 0, 2)).reshape(d, fw)
    xw, xa, y = _precompute(x, W_vgae1, W_assign, wef)
    spred2, s_f32, cnt = _vgae(adj, xw, W_vgae_s)

    src = jnp.concatenate([edge_index[0], jnp.zeros((ep - e,), I32)])
    dst = jnp.concatenate([edge_index[1], jnp.full((ep - e,), n, I32)])

    om = _omega_kernel(n, ep)(xa.reshape(-1), src, dst)
    h_pre = _scatter_kernel(n, fw, ep)(y, src, dst, om).reshape(n, fw)

    h_out, clog, lsum = _heads(h_pre, s_f32, W_demo, b_demo.reshape(1, 2))
    y_logits = _classifier(h_out, lsum, cnt, mask_param, W_cls1, b_cls1,
                           W_cls2, b_cls2)

    h = h_out.reshape(n, c_ch, hdim)
    s_pred = spred2[:, 0]
    return y_logits, h, clog, s_pred


# speedup vs baseline: 1.2587x; 1.1581x over previous
import functools

import jax
import jax.numpy as jnp
from jax import lax
from jax.experimental import pallas as pl
from jax.experimental.pallas import tpu as pltpu
from jax.experimental.pallas import tpu_sc as plsc

F32 = jnp.float32
I32 = jnp.int32


def _a_body(x_ref, wv1_ref, wa_ref, wef_ref, xw_ref, xa_ref, y_ref):
    xb = x_ref[...]
    xw_ref[...] = jnp.dot(xb, wv1_ref[...], preferred_element_type=F32)
    xa_ref[...] = jnp.dot(xb, wa_ref[...], preferred_element_type=F32)
    y_ref[...] = jnp.dot(xb, wef_ref[...], preferred_element_type=F32)


def _precompute(x, wv1, wa, wef):
    n, d = x.shape
    nb = 1000
    grid = n // nb
    return pl.pallas_call(
        _a_body,
        grid=(grid,),
        in_specs=[
            pl.BlockSpec((nb, d), lambda i: (i, 0)),
            pl.BlockSpec(wv1.shape, lambda i: (0, 0)),
            pl.BlockSpec(wa.shape, lambda i: (0, 0)),
            pl.BlockSpec(wef.shape, lambda i: (0, 0)),
        ],
        out_specs=[
            pl.BlockSpec((nb, wv1.shape[1]), lambda i: (i, 0)),
            pl.BlockSpec((nb, wa.shape[1]), lambda i: (i, 0)),
            pl.BlockSpec((nb, wef.shape[1]), lambda i: (i, 0)),
        ],
        out_shape=[
            jax.ShapeDtypeStruct((n, wv1.shape[1]), F32),
            jax.ShapeDtypeStruct((n, wa.shape[1]), F32),
            jax.ShapeDtypeStruct((n, wef.shape[1]), F32),
        ],
    )(x, wv1, wa, wef)


def _b_body(adj_ref, xw_ref, wvs_ref, sp_ref, sf_ref, cnt_ref):
    i = pl.program_id(0)
    ab = adj_ref[...]
    deg = jnp.sum(ab, axis=1, keepdims=True)
    an = ab / (deg + 1e-8)
    h1 = jnp.maximum(jnp.dot(an, xw_ref[...], preferred_element_type=F32), 0.0)
    sl = jnp.dot(h1, wvs_ref[...], preferred_element_type=F32)
    pred = sl[:, 1:2] > sl[:, 0:1]
    sp_ref[...] = pred.astype(I32)
    predf = pred.astype(F32)
    sf_ref[...] = predf

    @pl.when(i == 0)
    def _():
        cnt_ref[...] = jnp.zeros((1, 1), F32)

    cnt_ref[...] += jnp.sum(predf, keepdims=True)


def _vgae(adj, xw, wvs):
    n = adj.shape[0]
    nb = 200
    grid = n // nb
    return pl.pallas_call(
        _b_body,
        grid=(grid,),
        in_specs=[
            pl.BlockSpec((nb, n), lambda i: (i, 0)),
            pl.BlockSpec(xw.shape, lambda i: (0, 0)),
            pl.BlockSpec(wvs.shape, lambda i: (0, 0)),
        ],
        out_specs=[
            pl.BlockSpec((nb, 1), lambda i: (i, 0)),
            pl.BlockSpec((nb, 1), lambda i: (i, 0)),
            pl.BlockSpec((1, 1), lambda i: (0, 0)),
        ],
        out_shape=[
            jax.ShapeDtypeStruct((n, 1), I32),
            jax.ShapeDtypeStruct((n, 1), F32),
            jax.ShapeDtypeStruct((1, 1), F32),
        ],
    )(adj, xw, wvs)


def _omega_kernel(n_nodes, ep):
    ew = ep // 32
    mesh = plsc.VectorSubcoreMesh(core_axis_name="c", subcore_axis_name="s")

    @functools.partial(
        pl.kernel,
        out_type=jax.ShapeDtypeStruct((ep * 4,), F32),
        mesh=mesh,
        compiler_params=pltpu.CompilerParams(needs_layout_passes=False),
        scratch_types=[
            pltpu.VMEM((n_nodes * 4,), F32),
            pltpu.VMEM((ew,), I32),
            pltpu.VMEM((ew,), I32),
            pltpu.VMEM((ew * 4,), F32),
        ],
    )
    def k(xa_hbm, src_hbm, dst_hbm, om_hbm, xav, srcv, dstv, omv):
        wid = lax.axis_index("s") * 2 + lax.axis_index("c")
        base = wid * ew
        pltpu.sync_copy(xa_hbm, xav)
        pltpu.sync_copy(src_hbm.at[pl.ds(base, ew)], srcv)
        pltpu.sync_copy(dst_hbm.at[pl.ds(base, ew)], dstv)
        lane = lax.iota(I32, 16)
        nmax = jnp.full((16,), n_nodes - 1, I32)

        def chunk(i, carry):
            sv = jnp.minimum(srcv[pl.ds(i * 16, 16)], nmax) * 4
            dv = jnp.minimum(dstv[pl.ds(i * 16, 16)], nmax) * 4
            z = []
            for c in range(4):
                colc = jnp.full((16,), c, I32)
                za = plsc.load_gather(xav, [sv + colc])
                zb = plsc.load_gather(xav, [dv + colc])
                z.append(za + zb)
            m = jnp.maximum(jnp.maximum(z[0], z[1]), jnp.maximum(z[2], z[3]))
            es = [jnp.exp(zc - m) for zc in z]
            tot = es[0] + es[1] + es[2] + es[3]
            inv = 1.0 / tot
            eidx = (jnp.full((16,), i * 16, I32) + lane) * 4
            for c in range(4):
                plsc.store_scatter(omv, [eidx + c], es[c] * inv)
            return carry

        lax.fori_loop(0, ew // 16, chunk, 0)
        pltpu.sync_copy(omv, om_hbm.at[pl.ds(base * 4, ew * 4)])

    return k


def _scatter_kernel(n_nodes, fw, ep):
    rpt = 320
    acc_rows = 328
    fb = 64
    cap = 128
    sck = 2048
    now = ep // sck
    mesh = plsc.VectorSubcoreMesh(core_axis_name="c", subcore_axis_name="s")

    @functools.partial(
        pl.kernel,
        out_type=jax.ShapeDtypeStruct((n_nodes * fw,), F32),
        mesh=mesh,
        compiler_params=pltpu.CompilerParams(needs_layout_passes=False),
        scratch_types=[
            pltpu.VMEM((acc_rows * fw,), F32),
            pltpu.VMEM((2, fb, fw), F32),
            pltpu.VMEM((2, sck), I32),
            pltpu.VMEM((2, sck), I32),
            pltpu.VMEM((cap,), I32),
            pltpu.VMEM((cap,), I32),
            pltpu.VMEM((cap,), I32),
            pltpu.VMEM((2, fb), I32),
            pltpu.VMEM((2 * fb,), I32),
            pltpu.VMEM((2, 4, fb), I32),
            pltpu.VMEM((2 * 4 * fb,), F32),
            pltpu.SemaphoreType.DMA,
            pltpu.SemaphoreType.DMA,
            pltpu.SemaphoreType.DMA,
            pltpu.SemaphoreType.DMA,
            pltpu.SemaphoreType.DMA,
        ],
    )
    def k(y_hbm, src_hbm, dst_hbm, om_hbm, out_hbm, acc, gbuf, srcw, dstw,
          csrc, clv, cpe, csrcP, clvP, omidxP, obufP, semY0, semY1, semO0,
          semO1, sems):
        w = lax.axis_index("s") * 2 + lax.axis_index("c")
        base_row = w * rpt
        lane = lax.iota(I32, 16)
        zero16 = jnp.zeros((16,), F32)
        trash16 = jnp.full((16,), rpt, I32) + (lane & 3)
        semY = (semY0, semY1)
        semO = (semO0, semO1)
        qs = fb // 4

        def zrow(i, carry):
            base = i * fw
            for v in range(fw // 16):
                acc[pl.ds(base + v * 16, 16)] = zero16
            return carry

        lax.fori_loop(0, acc_rows, zrow, 0)

        def submit(P):
            for q in range(fb // 16):
                sl = pl.ds(q * 16, 16)
                pe4 = cpe[sl] * 4
                csrcP[P, sl] = csrc[sl]
                clvP[pl.ds(P * fb + q * 16, 16)] = clv[sl]
                for ch in range(4):
                    omidxP[P, ch, sl] = pe4 + ch
            for q in range(4):
                pltpu.async_copy(
                    y_hbm.at[csrcP.at[P, pl.ds(q * qs, qs)]],
                    gbuf.at[P, pl.ds(q * qs, qs)], semY[P])
            for ch in range(4):
                pltpu.async_copy(om_hbm.at[omidxP.at[P, ch]],
                                 obufP.at[pl.ds((P * 4 + ch) * fb, fb)],
                                 semO[P])

        def process(P):
            for q in range(4):
                pltpu.make_async_copy(
                    y_hbm.at[csrcP.at[P, pl.ds(q * qs, qs)]],
                    gbuf.at[P, pl.ds(q * qs, qs)], semY[P]).wait()
            for ch in range(4):
                pltpu.make_async_copy(om_hbm.at[omidxP.at[P, ch]],
                                      obufP.at[pl.ds((P * 4 + ch) * fb, fb)],
                                      semO[P]).wait()

            def fedge(e, cy):
                ev = jnp.full((16,), e, I32)
                rowv = plsc.load_gather(clvP, [ev + (P * fb)])
                rowbase = rowv * fw + lane
                for ch in range(4):
                    wv = plsc.load_gather(obufP, [ev + ((P * 4 + ch) * fb)])
                    for v in range(4):
                        off = ch * 64 + v * 16
                        val = gbuf[P, e, pl.ds(off, 16)] * wv
                        plsc.addupdate_scatter(acc, [rowbase + off], val)
                return cy

            lax.fori_loop(0, fb, fedge, 0, unroll=4)

        def fill_trash(base):
            for kk in range(fb // 16):
                sl = pl.ds(base + kk * 16, 16)
                csrc[sl] = lane * 0
                clv[sl] = trash16
                cpe[sl] = lane * 0

        fill_trash(0)
        submit(0)

        pltpu.async_copy(src_hbm.at[pl.ds(0, sck)], srcw.at[0], sems)
        pltpu.async_copy(dst_hbm.at[pl.ds(0, sck)], dstw.at[0], sems)

        def window(o, carry):
            eb = o * sck
            par_w = o % 2
            pltpu.make_async_copy(src_hbm.at[pl.ds(eb, sck)],
                                  srcw.at[par_w], sems).wait()
            pltpu.make_async_copy(dst_hbm.at[pl.ds(eb, sck)],
                                  dstw.at[par_w], sems).wait()

            @pl.when(o + 1 < now)
            def _():
                nxt = (o + 1) % 2
                eb2 = (o + 1) * sck
                pltpu.async_copy(src_hbm.at[pl.ds(eb2, sck)],
                                 srcw.at[nxt], sems)
                pltpu.async_copy(dst_hbm.at[pl.ds(eb2, sck)],
                                 dstw.at[nxt], sems)

            def block(b, pc):
                ptr_in, par_in = pc
                datas = []
                cnts = []
                for q in range(4):
                    j_base = b * 64 + q * 16
                    s16 = srcw[par_w, pl.ds(j_base, 16)]
                    d16 = dstw[par_w, pl.ds(j_base, 16)]
                    lv = d16 - base_row
                    mask = (lv >= 0) & (lv < rpt)
                    pe = jnp.full((16,), eb, I32) + (j_base + lane)
                    datas.append((s16, lv, pe, mask))
                    cnts.append(plsc.all_reduce_population_count(mask)[0])
                p = ptr_in
                for q in range(4):
                    s16, lv, pe, mask = datas[q]
                    plsc.store_compressed(csrc.at[pl.ds(p, 16)], s16,
                                          mask=mask)
                    plsc.store_compressed(clv.at[pl.ds(p, 16)], lv,
                                          mask=mask)
                    plsc.store_compressed(cpe.at[pl.ds(p, 16)], pe,
                                          mask=mask)
                    p = p + cnts[q]

                @pl.when(p >= fb)
                def _():
                    @pl.when(par_in == 0)
                    def _():
                        submit(0)
                        process(1)

                    @pl.when(par_in == 1)
                    def _():
                        submit(1)
                        process(0)

                    for ref in (csrc, clv, cpe):
                        for q2 in range(fb // 16):
                            ref[pl.ds(q2 * 16, 16)] = \
                                ref[pl.ds(fb + q2 * 16, 16)]

                newp = lax.select(p >= fb, p - fb, p)
                newpar = lax.select(p >= fb, 1 - par_in, par_in)
                return (newp, newpar)

            return lax.fori_loop(0, sck // 64, block, carry)

        ptr, par = lax.fori_loop(0, now, window, (0, 1))

        @pl.when(par == 0)
        def _():
            process(1)

        @pl.when(par == 1)
        def _():
            process(0)

        fill_trash(ptr)

        @pl.when(par == 0)
        def _():
            submit(0)
            process(0)

        @pl.when(par == 1)
        def _():
            submit(1)
            process(1)

        @pl.when(w < 31)
        def _():
            pltpu.sync_copy(acc.at[pl.ds(0, rpt * fw)],
                            out_hbm.at[pl.ds(base_row * fw, rpt * fw)])

        @pl.when(w == 31)
        def _():
            last = (n_nodes - 31 * rpt) * fw
            pltpu.sync_copy(acc.at[pl.ds(0, last)],
                            out_hbm.at[pl.ds(base_row * fw, last)])

    return k


def _e_body(hp_ref, sf_ref, wd_ref, bd_ref, h_ref, clog_ref, lsum_ref):
    i = pl.program_id(0)
    hb = jnp.maximum(hp_ref[...], 0.0)
    h_ref[...] = hb
    wd = wd_ref[...]
    bd = bd_ref[...]
    s = sf_ref[...]
    parts = []
    for c in range(4):
        hc = hb[:, c * 64:(c + 1) * 64]
        lg = jnp.dot(hc, wd, preferred_element_type=F32) + bd
        clog_ref[c, :, :] = lg
        l0 = lg[:, 0:1]
        l1 = lg[:, 1:2]
        m = jnp.maximum(l0, l1)
        lse = m + jnp.log(jnp.exp(l0 - m) + jnp.exp(l1 - m))
        picked = jnp.where(s > 0.5, l1, l0)
        parts.append(jnp.sum(lse - picked, axis=0, keepdims=True))
    part = jnp.concatenate(parts, axis=1)

    @pl.when(i == 0)
    def _():
        lsum_ref[...] = jnp.zeros((1, 4), F32)

    lsum_ref[...] += part


def _heads(h_pre, s_f32, wd, bd):
    n, fw = h_pre.shape
    nb = 200
    grid = n // nb
    return pl.pallas_call(
        _e_body,
        grid=(grid,),
        in_specs=[
            pl.BlockSpec((nb, fw), lambda i: (i, 0)),
            pl.BlockSpec((nb, 1), lambda i: (i, 0)),
            pl.BlockSpec(wd.shape, lambda i: (0, 0)),
            pl.BlockSpec((1, 2), lambda i: (0, 0)),
        ],
        out_specs=[
            pl.BlockSpec((nb, fw), lambda i: (i, 0)),
            pl.BlockSpec((4, nb, 2), lambda i: (0, i, 0)),
            pl.BlockSpec((1, 4), lambda i: (0, 0)),
        ],
        out_shape=[
            jax.ShapeDtypeStruct((n, fw), F32),
            jax.ShapeDtypeStruct((4, n, 2), F32),
            jax.ShapeDtypeStruct((1, 4), F32),
        ],
    )(h_pre, s_f32, wd, bd)


def _f_body(h_ref, lsum_ref, cnt_ref, mp_ref, wc1_ref, bc1_ref, wc2_ref,
            bc2_ref, y_ref):
    n_total = 10000.0
    cl = lsum_ref[...] / n_total
    cw = cnt_ref[...]
    p1 = cw / n_total
    p0 = 1.0 - p1
    hs = -(p0 * jnp.log(p0 + 1e-12) + p1 * jnp.log(p1 + 1e-12))
    cs = jnp.clip(1.0 - cl / (hs + 1e-8), 0.0, 1.0)
    gate = jax.nn.sigmoid(mp_ref[...])
    hb = h_ref[...]
    acc = None
    for c in range(4):
        factor = 1.0 - cs[0, c] * gate[c:c + 1, :]
        hf = hb[:, c * 64:(c + 1) * 64] * factor
        t = jnp.dot(hf, wc1_ref[c], preferred_element_type=F32)
        acc = t if acc is None else acc + t
    t1 = jnp.maximum(acc + bc1_ref[...], 0.0)
    y_ref[...] = jnp.dot(t1, wc2_ref[...], preferred_element_type=F32) \
        + bc2_ref[...]


def _classifier(h, lsum, cnt, mp, wc1, bc1, wc2, bc2):
    n, fw = h.shape
    nb = 200
    grid = n // nb
    wc1r = wc1.reshape(4, 64, 64)
    return pl.pallas_call(
        _f_body,
        grid=(grid,),
        in_specs=[
            pl.BlockSpec((nb, fw), lambda i: (i, 0)),
            pl.BlockSpec((1, 4), lambda i: (0, 0)),
            pl.BlockSpec((1, 1), lambda i: (0, 0)),
            pl.BlockSpec(mp.shape, lambda i: (0, 0)),
            pl.BlockSpec(wc1r.shape, lambda i: (0, 0, 0)),
            pl.BlockSpec((1, 64), lambda i: (0, 0)),
            pl.BlockSpec(wc2.shape, lambda i: (0, 0)),
            pl.BlockSpec((1, 2), lambda i: (0, 0)),
        ],
        out_specs=pl.BlockSpec((nb, 2), lambda i: (i, 0)),
        out_shape=jax.ShapeDtypeStruct((n, 2), F32),
    )(h, lsum, cnt, mp, wc1r, bc1.reshape(1, 64), wc2, bc2.reshape(1, 2))


def kernel(adj, x, edge_index, W_vgae1, W_vgae_s, W_assign, W_enc, W_demo,
           b_demo, mask_param, W_cls1, b_cls1, W_cls2, b_cls2):
    n, d = x.shape
    c_ch, _, hdim = W_enc.shape
    fw = c_ch * hdim
    e = edge_index.shape[1]
    ep = ((e + 4095) // 4096) * 4096

    wef = jnp.transpose(W_enc, (1, 0, 2)).reshape(d, fw)
    xw, xa, y = _precompute(x, W_vgae1, W_assign, wef)
    spred2, s_f32, cnt = _vgae(adj, xw, W_vgae_s)

    src = jnp.concatenate([edge_index[0], jnp.zeros((ep - e,), I32)])
    dst = jnp.concatenate([edge_index[1], jnp.full((ep - e,), n, I32)])

    om = _omega_kernel(n, ep)(xa.reshape(-1), src, dst)
    h_pre = _scatter_kernel(n, fw, ep)(y, src, dst, om).reshape(n, fw)

    h_out, clog, lsum = _heads(h_pre, s_f32, W_demo, b_demo.reshape(1, 2))
    y_logits = _classifier(h_out, lsum, cnt, mask_param, W_cls1, b_cls1,
                           W_cls2, b_cls2)

    h = h_out.reshape(n, c_ch, hdim)
    s_pred = spred2[:, 0]
    return y_logits, h, clog, s_pred


# speedup vs baseline: 1.2921x; 1.0265x over previous
import functools

import jax
import jax.numpy as jnp
from jax import lax
from jax.experimental import pallas as pl
from jax.experimental.pallas import tpu as pltpu
from jax.experimental.pallas import tpu_sc as plsc

F32 = jnp.float32
I32 = jnp.int32


def _a_body(x_ref, wv1_ref, wa_ref, wef_ref, xw_ref, xa_ref, y_ref):
    xb = x_ref[...]
    xw_ref[...] = jnp.dot(xb, wv1_ref[...], preferred_element_type=F32)
    xa_ref[...] = jnp.dot(xb, wa_ref[...], preferred_element_type=F32)
    y_ref[...] = jnp.dot(xb, wef_ref[...], preferred_element_type=F32)


def _precompute(x, wv1, wa, wef):
    n, d = x.shape
    nb = 1000
    grid = n // nb
    return pl.pallas_call(
        _a_body,
        grid=(grid,),
        in_specs=[
            pl.BlockSpec((nb, d), lambda i: (i, 0)),
            pl.BlockSpec(wv1.shape, lambda i: (0, 0)),
            pl.BlockSpec(wa.shape, lambda i: (0, 0)),
            pl.BlockSpec(wef.shape, lambda i: (0, 0)),
        ],
        out_specs=[
            pl.BlockSpec((nb, wv1.shape[1]), lambda i: (i, 0)),
            pl.BlockSpec((nb, wa.shape[1]), lambda i: (i, 0)),
            pl.BlockSpec((nb, wef.shape[1]), lambda i: (i, 0)),
        ],
        out_shape=[
            jax.ShapeDtypeStruct((n, wv1.shape[1]), F32),
            jax.ShapeDtypeStruct((n, wa.shape[1]), F32),
            jax.ShapeDtypeStruct((n, wef.shape[1]), F32),
        ],
    )(x, wv1, wa, wef)


def _b_body(adj_ref, xw_ref, wvs_ref, sp_ref, sf_ref, cnt_ref):
    i = pl.program_id(0)
    ab = adj_ref[...]
    deg = jnp.sum(ab, axis=1, keepdims=True)
    an = ab / (deg + 1e-8)
    h1 = jnp.maximum(jnp.dot(an, xw_ref[...], preferred_element_type=F32), 0.0)
    sl = jnp.dot(h1, wvs_ref[...], preferred_element_type=F32)
    pred = sl[:, 1:2] > sl[:, 0:1]
    sp_ref[...] = pred.astype(I32)
    predf = pred.astype(F32)
    sf_ref[...] = predf

    @pl.when(i == 0)
    def _():
        cnt_ref[...] = jnp.zeros((1, 1), F32)

    cnt_ref[...] += jnp.sum(predf, keepdims=True)


def _vgae(adj, xw, wvs):
    n = adj.shape[0]
    nb = 200
    grid = n // nb
    return pl.pallas_call(
        _b_body,
        grid=(grid,),
        in_specs=[
            pl.BlockSpec((nb, n), lambda i: (i, 0)),
            pl.BlockSpec(xw.shape, lambda i: (0, 0)),
            pl.BlockSpec(wvs.shape, lambda i: (0, 0)),
        ],
        out_specs=[
            pl.BlockSpec((nb, 1), lambda i: (i, 0)),
            pl.BlockSpec((nb, 1), lambda i: (i, 0)),
            pl.BlockSpec((1, 1), lambda i: (0, 0)),
        ],
        out_shape=[
            jax.ShapeDtypeStruct((n, 1), I32),
            jax.ShapeDtypeStruct((n, 1), F32),
            jax.ShapeDtypeStruct((1, 1), F32),
        ],
    )(adj, xw, wvs)


def _omega_kernel(n_nodes, ep):
    ew = ep // 32
    mesh = plsc.VectorSubcoreMesh(core_axis_name="c", subcore_axis_name="s")

    @functools.partial(
        pl.kernel,
        out_type=jax.ShapeDtypeStruct((ep * 4,), F32),
        mesh=mesh,
        compiler_params=pltpu.CompilerParams(needs_layout_passes=False),
        scratch_types=[
            pltpu.VMEM((n_nodes * 4,), F32),
            pltpu.VMEM((ew,), I32),
            pltpu.VMEM((ew,), I32),
            pltpu.VMEM((ew * 4,), F32),
        ],
    )
    def k(xa_hbm, src_hbm, dst_hbm, om_hbm, xav, srcv, dstv, omv):
        wid = lax.axis_index("s") * 2 + lax.axis_index("c")
        base = wid * ew
        pltpu.sync_copy(xa_hbm, xav)
        pltpu.sync_copy(src_hbm.at[pl.ds(base, ew)], srcv)
        pltpu.sync_copy(dst_hbm.at[pl.ds(base, ew)], dstv)
        lane = lax.iota(I32, 16)
        nmax = jnp.full((16,), n_nodes - 1, I32)

        def chunk(i, carry):
            sv = jnp.minimum(srcv[pl.ds(i * 16, 16)], nmax) * 4
            dv = jnp.minimum(dstv[pl.ds(i * 16, 16)], nmax) * 4
            z = []
            for c in range(4):
                colc = jnp.full((16,), c, I32)
                za = plsc.load_gather(xav, [sv + colc])
                zb = plsc.load_gather(xav, [dv + colc])
                z.append(za + zb)
            m = jnp.maximum(jnp.maximum(z[0], z[1]), jnp.maximum(z[2], z[3]))
            es = [jnp.exp(zc - m) for zc in z]
            tot = es[0] + es[1] + es[2] + es[3]
            inv = 1.0 / tot
            eidx = (jnp.full((16,), i * 16, I32) + lane) * 4
            for c in range(4):
                plsc.store_scatter(omv, [eidx + c], es[c] * inv)
            return carry

        lax.fori_loop(0, ew // 16, chunk, 0)
        pltpu.sync_copy(omv, om_hbm.at[pl.ds(base * 4, ew * 4)])

    return k


def _scatter_kernel(n_nodes, fw, ep):
    rpt = 320
    acc_rows = 328
    fb = 64
    cap = 128
    sck = 2048
    now = ep // sck
    mesh = plsc.VectorSubcoreMesh(core_axis_name="c", subcore_axis_name="s")

    @functools.partial(
        pl.kernel,
        out_type=jax.ShapeDtypeStruct((n_nodes * fw,), F32),
        mesh=mesh,
        compiler_params=pltpu.CompilerParams(needs_layout_passes=False),
        scratch_types=[
            pltpu.VMEM((acc_rows * fw,), F32),
            pltpu.VMEM((2, fb, fw), F32),
            pltpu.VMEM((2, sck), I32),
            pltpu.VMEM((2, sck), I32),
            pltpu.VMEM((cap,), I32),
            pltpu.VMEM((cap,), I32),
            pltpu.VMEM((cap,), I32),
            pltpu.VMEM((2, fb), I32),
            pltpu.VMEM((2 * fb,), I32),
            pltpu.VMEM((2, 4, fb), I32),
            pltpu.VMEM((2 * 4 * fb,), F32),
            pltpu.SemaphoreType.DMA,
            pltpu.SemaphoreType.DMA,
            pltpu.SemaphoreType.DMA,
            pltpu.SemaphoreType.DMA,
            pltpu.SemaphoreType.DMA,
        ],
    )
    def k(y_hbm, src_hbm, dst_hbm, om_hbm, out_hbm, acc, gbuf, srcw, dstw,
          csrc, clv, cpe, csrcP, clvP, omidxP, obufP, semY0, semY1, semO0,
          semO1, sems):
        w = lax.axis_index("s") * 2 + lax.axis_index("c")
        base_row = w * rpt
        lane = lax.iota(I32, 16)
        zero16 = jnp.zeros((16,), F32)
        trash16 = jnp.full((16,), rpt, I32) + (lane & 3)
        semY = (semY0, semY1)
        semO = (semO0, semO1)
        qs = fb // 4

        def zrow(i, carry):
            base = i * fw
            for v in range(fw // 16):
                acc[pl.ds(base + v * 16, 16)] = zero16
            return carry

        lax.fori_loop(0, acc_rows, zrow, 0)

        def submit(P):
            for q in range(fb // 16):
                sl = pl.ds(q * 16, 16)
                pe4 = cpe[sl] * 4
                csrcP[P, sl] = csrc[sl]
                clvP[pl.ds(P * fb + q * 16, 16)] = clv[sl]
                for ch in range(4):
                    omidxP[P, ch, sl] = pe4 + ch
            for q in range(4):
                pltpu.async_copy(
                    y_hbm.at[csrcP.at[P, pl.ds(q * qs, qs)]],
                    gbuf.at[P, pl.ds(q * qs, qs)], semY[P])
            for ch in range(4):
                pltpu.async_copy(om_hbm.at[omidxP.at[P, ch]],
                                 obufP.at[pl.ds((P * 4 + ch) * fb, fb)],
                                 semO[P])

        def process(P):
            for q in range(4):
                pltpu.make_async_copy(
                    y_hbm.at[csrcP.at[P, pl.ds(q * qs, qs)]],
                    gbuf.at[P, pl.ds(q * qs, qs)], semY[P]).wait()
            for ch in range(4):
                pltpu.make_async_copy(om_hbm.at[omidxP.at[P, ch]],
                                      obufP.at[pl.ds((P * 4 + ch) * fb, fb)],
                                      semO[P]).wait()

            def fedge(e, cy):
                ev = jnp.full((16,), e, I32)
                rowv = plsc.load_gather(clvP, [ev + (P * fb)])
                rowbase = rowv * fw + lane
                for ch in range(4):
                    wv = plsc.load_gather(obufP, [ev + ((P * 4 + ch) * fb)])
                    for v in range(4):
                        off = ch * 64 + v * 16
                        val = gbuf[P, e, pl.ds(off, 16)] * wv
                        plsc.addupdate_scatter(acc, [rowbase + off], val)
                return cy

            lax.fori_loop(0, fb, fedge, 0, unroll=8)

        def fill_trash(base):
            for kk in range(fb // 16):
                sl = pl.ds(base + kk * 16, 16)
                csrc[sl] = lane * 0
                clv[sl] = trash16
                cpe[sl] = lane * 0

        fill_trash(0)
        submit(0)

        pltpu.async_copy(src_hbm.at[pl.ds(0, sck)], srcw.at[0], sems)
        pltpu.async_copy(dst_hbm.at[pl.ds(0, sck)], dstw.at[0], sems)

        def window(o, carry):
            eb = o * sck
            par_w = o % 2
            pltpu.make_async_copy(src_hbm.at[pl.ds(eb, sck)],
                                  srcw.at[par_w], sems).wait()
            pltpu.make_async_copy(dst_hbm.at[pl.ds(eb, sck)],
                                  dstw.at[par_w], sems).wait()

            @pl.when(o + 1 < now)
            def _():
                nxt = (o + 1) % 2
                eb2 = (o + 1) * sck
                pltpu.async_copy(src_hbm.at[pl.ds(eb2, sck)],
                                 srcw.at[nxt], sems)
                pltpu.async_copy(dst_hbm.at[pl.ds(eb2, sck)],
                                 dstw.at[nxt], sems)

            def block(b, pc):
                ptr_in, par_in = pc
                datas = []
                cnts = []
                for q in range(4):
                    j_base = b * 64 + q * 16
                    s16 = srcw[par_w, pl.ds(j_base, 16)]
                    d16 = dstw[par_w, pl.ds(j_base, 16)]
                    lv = d16 - base_row
                    mask = lv.astype(jnp.uint32) < jnp.uint32(rpt)
                    pe = jnp.full((16,), eb, I32) + (j_base + lane)
                    datas.append((s16, lv, pe, mask))
                    cnts.append(plsc.all_reduce_population_count(mask)[0])
                p = ptr_in
                for q in range(4):
                    s16, lv, pe, mask = datas[q]
                    plsc.store_compressed(csrc.at[pl.ds(p, 16)], s16,
                                          mask=mask)
                    plsc.store_compressed(clv.at[pl.ds(p, 16)], lv,
                                          mask=mask)
                    plsc.store_compressed(cpe.at[pl.ds(p, 16)], pe,
                                          mask=mask)
                    p = p + cnts[q]

                @pl.when(p >= fb)
                def _():
                    @pl.when(par_in == 0)
                    def _():
                        submit(0)
                        process(1)

                    @pl.when(par_in == 1)
                    def _():
                        submit(1)
                        process(0)

                    for ref in (csrc, clv, cpe):
                        for q2 in range(fb // 16):
                            ref[pl.ds(q2 * 16, 16)] = \
                                ref[pl.ds(fb + q2 * 16, 16)]

                newp = lax.select(p >= fb, p - fb, p)
                newpar = lax.select(p >= fb, 1 - par_in, par_in)
                return (newp, newpar)

            return lax.fori_loop(0, sck // 64, block, carry)

        ptr, par = lax.fori_loop(0, now, window, (0, 1))

        @pl.when(par == 0)
        def _():
            process(1)

        @pl.when(par == 1)
        def _():
            process(0)

        fill_trash(ptr)

        @pl.when(par == 0)
        def _():
            submit(0)
            process(0)

        @pl.when(par == 1)
        def _():
            submit(1)
            process(1)

        @pl.when(w < 31)
        def _():
            pltpu.sync_copy(acc.at[pl.ds(0, rpt * fw)],
                            out_hbm.at[pl.ds(base_row * fw, rpt * fw)])

        @pl.when(w == 31)
        def _():
            last = (n_nodes - 31 * rpt) * fw
            pltpu.sync_copy(acc.at[pl.ds(0, last)],
                            out_hbm.at[pl.ds(base_row * fw, last)])

    return k


def _e_body(hp_ref, sf_ref, wd_ref, bd_ref, h_ref, clog_ref, lsum_ref):
    i = pl.program_id(0)
    hb = jnp.maximum(hp_ref[...], 0.0)
    h_ref[...] = hb
    wd = wd_ref[...]
    bd = bd_ref[...]
    s = sf_ref[...]
    parts = []
    for c in range(4):
        hc = hb[:, c * 64:(c + 1) * 64]
        lg = jnp.dot(hc, wd, preferred_element_type=F32) + bd
        clog_ref[c, :, :] = lg
        l0 = lg[:, 0:1]
        l1 = lg[:, 1:2]
        m = jnp.maximum(l0, l1)
        lse = m + jnp.log(jnp.exp(l0 - m) + jnp.exp(l1 - m))
        picked = jnp.where(s > 0.5, l1, l0)
        parts.append(jnp.sum(lse - picked, axis=0, keepdims=True))
    part = jnp.concatenate(parts, axis=1)

    @pl.when(i == 0)
    def _():
        lsum_ref[...] = jnp.zeros((1, 4), F32)

    lsum_ref[...] += part


def _heads(h_pre, s_f32, wd, bd):
    n, fw = h_pre.shape
    nb = 200
    grid = n // nb
    return pl.pallas_call(
        _e_body,
        grid=(grid,),
        in_specs=[
            pl.BlockSpec((nb, fw), lambda i: (i, 0)),
            pl.BlockSpec((nb, 1), lambda i: (i, 0)),
            pl.BlockSpec(wd.shape, lambda i: (0, 0)),
            pl.BlockSpec((1, 2), lambda i: (0, 0)),
        ],
        out_specs=[
            pl.BlockSpec((nb, fw), lambda i: (i, 0)),
            pl.BlockSpec((4, nb, 2), lambda i: (0, i, 0)),
            pl.BlockSpec((1, 4), lambda i: (0, 0)),
        ],
        out_shape=[
            jax.ShapeDtypeStruct((n, fw), F32),
            jax.ShapeDtypeStruct((4, n, 2), F32),
            jax.ShapeDtypeStruct((1, 4), F32),
        ],
    )(h_pre, s_f32, wd, bd)


def _f_body(h_ref, lsum_ref, cnt_ref, mp_ref, wc1_ref, bc1_ref, wc2_ref,
            bc2_ref, y_ref):
    n_total = 10000.0
    cl = lsum_ref[...] / n_total
    cw = cnt_ref[...]
    p1 = cw / n_total
    p0 = 1.0 - p1
    hs = -(p0 * jnp.log(p0 + 1e-12) + p1 * jnp.log(p1 + 1e-12))
    cs = jnp.clip(1.0 - cl / (hs + 1e-8), 0.0, 1.0)
    gate = jax.nn.sigmoid(mp_ref[...])
    hb = h_ref[...]
    acc = None
    for c in range(4):
        factor = 1.0 - cs[0, c] * gate[c:c + 1, :]
        hf = hb[:, c * 64:(c + 1) * 64] * factor
        t = jnp.dot(hf, wc1_ref[c], preferred_element_type=F32)
        acc = t if acc is None else acc + t
    t1 = jnp.maximum(acc + bc1_ref[...], 0.0)
    y_ref[...] = jnp.dot(t1, wc2_ref[...], preferred_element_type=F32) \
        + bc2_ref[...]


def _classifier(h, lsum, cnt, mp, wc1, bc1, wc2, bc2):
    n, fw = h.shape
    nb = 200
    grid = n // nb
    wc1r = wc1.reshape(4, 64, 64)
    return pl.pallas_call(
        _f_body,
        grid=(grid,),
        in_specs=[
            pl.BlockSpec((nb, fw), lambda i: (i, 0)),
            pl.BlockSpec((1, 4), lambda i: (0, 0)),
            pl.BlockSpec((1, 1), lambda i: (0, 0)),
            pl.BlockSpec(mp.shape, lambda i: (0, 0)),
            pl.BlockSpec(wc1r.shape, lambda i: (0, 0, 0)),
            pl.BlockSpec((1, 64), lambda i: (0, 0)),
            pl.BlockSpec(wc2.shape, lambda i: (0, 0)),
            pl.BlockSpec((1, 2), lambda i: (0, 0)),
        ],
        out_specs=pl.BlockSpec((nb, 2), lambda i: (i, 0)),
        out_shape=jax.ShapeDtypeStruct((n, 2), F32),
    )(h, lsum, cnt, mp, wc1r, bc1.reshape(1, 64), wc2, bc2.reshape(1, 2))


def kernel(adj, x, edge_index, W_vgae1, W_vgae_s, W_assign, W_enc, W_demo,
           b_demo, mask_param, W_cls1, b_cls1, W_cls2, b_cls2):
    n, d = x.shape
    c_ch, _, hdim = W_enc.shape
    fw = c_ch * hdim
    e = edge_index.shape[1]
    ep = ((e + 4095) // 4096) * 4096

    wef = jnp.transpose(W_enc, (1, 0, 2)).reshape(d, fw)
    xw, xa, y = _precompute(x, W_vgae1, W_assign, wef)
    spred2, s_f32, cnt = _vgae(adj, xw, W_vgae_s)

    src = jnp.concatenate([edge_index[0], jnp.zeros((ep - e,), I32)])
    dst = jnp.concatenate([edge_index[1], jnp.full((ep - e,), n, I32)])

    om = _omega_kernel(n, ep)(xa.reshape(-1), src, dst)
    h_pre = _scatter_kernel(n, fw, ep)(y, src, dst, om).reshape(n, fw)

    h_out, clog, lsum = _heads(h_pre, s_f32, W_demo, b_demo.reshape(1, 2))
    y_logits = _classifier(h_out, lsum, cnt, mask_param, W_cls1, b_cls1,
                           W_cls2, b_cls2)

    h = h_out.reshape(n, c_ch, hdim)
    s_pred = spred2[:, 0]
    return y_logits, h, clog, s_pred
